# baseline (speedup 1.0000x reference)
import jax
import jax.numpy as jnp
from jax import lax
from jax.experimental import pallas as pl
from jax.experimental.pallas import tpu as pltpu
from jax.experimental.pallas import tpu_sc as plsc

_INFO = plsc.get_sparse_core_info()
_NC, _NS = _INFO.num_cores, _INFO.num_subcores
_NW = _NC * _NS

_B = 4
_CW = 2048
_E = 1024
_ROWS = _B * _CW
_TPW = _CW // _NW
_C = 32
_NP = _TPW // _C
_LANE = 16
_COLS = _E // _LANE


def _emb_body(x_hbm, wte_hbm, wpe_hbm, out_hbm,
              pos_v, tok0_v, tok1_v, idx0_v, idx1_v,
              sg0, sg1, ss0, ss1):
    wid = lax.axis_index("s") * _NC + lax.axis_index("c")
    t0 = wid * _TPW
    toks = (tok0_v, tok1_v)
    idxs = (idx0_v, idx1_v)
    gsems = (sg0, sg1)
    ssems = (ss0, ss1)
    chunks = [(p, b) for p in range(_NP) for b in range(_B)]
    n = len(chunks)

    def start_gather(i):
        p, b = chunks[i]
        s = i % 2
        pltpu.sync_copy(x_hbm.at[pl.ds(b * _CW + t0 + p * _C, _C)], idxs[s])
        return pltpu.async_copy(wte_hbm.at[idxs[s]], toks[s], gsems[s])

    pltpu.sync_copy(wpe_hbm.at[pl.ds(t0, _C)], pos_v)
    gather = start_gather(0)
    scatters = [None, None]
    for i in range(n):
        p, b = chunks[i]
        s = i % 2
        nxt = (i + 1) % 2
        if i + 1 < n:
            if scatters[nxt] is not None:
                scatters[nxt].wait()
                scatters[nxt] = None
            next_gather = start_gather(i + 1)
        gather.wait()
        if i + 1 < n:
            gather = next_gather
        if p > 0 and b == 0:
            pltpu.sync_copy(wpe_hbm.at[pl.ds(t0 + p * _C, _C)], pos_v)
        tok = toks[s]

        def add_row(r, _, tok=tok):
            for j in range(_COLS):
                plsc.addupdate(tok.at[r, pl.ds(j * _LANE, _LANE)],
                               pos_v[r, pl.ds(j * _LANE, _LANE)])
            return 0

        lax.fori_loop(0, _C, add_row, 0)
        row0 = b * _CW + t0 + p * _C
        scatters[s] = pltpu.async_copy(tok, out_hbm.at[pl.ds(row0, _C)],
                                       ssems[s])
    for d in scatters:
        if d is not None:
            d.wait()


@jax.jit
def _emb(x_flat, wte, wpe):
    mesh = plsc.VectorSubcoreMesh(core_axis_name="c", subcore_axis_name="s")
    return pl.kernel(
        _emb_body,
        out_type=jax.ShapeDtypeStruct((_ROWS, _E), jnp.float32),
        mesh=mesh,
        scratch_types=[
            pltpu.VMEM((_C, _E), jnp.float32),
            pltpu.VMEM((_C, _E), jnp.float32),
            pltpu.VMEM((_C, _E), jnp.float32),
            pltpu.VMEM((_C,), jnp.int32),
            pltpu.VMEM((_C,), jnp.int32),
            pltpu.SemaphoreType.DMA,
            pltpu.SemaphoreType.DMA,
            pltpu.SemaphoreType.DMA,
            pltpu.SemaphoreType.DMA,
        ],
    )(x_flat, wte, wpe)


def kernel(x, wte, wpe, pos):
    del pos
    x_flat = x.reshape(_ROWS).astype(jnp.int32)
    out = _emb(x_flat, wte, wpe)
    return out.reshape(_B, _CW, _E)


# speedup vs baseline: 1.3652x; 1.3652x over previous
import jax
import jax.numpy as jnp
from jax import lax
from jax.experimental import pallas as pl
from jax.experimental.pallas import tpu as pltpu
from jax.experimental.pallas import tpu_sc as plsc

_INFO = plsc.get_sparse_core_info()
_NC, _NS = _INFO.num_cores, _INFO.num_subcores
_NW = _NC * _NS

_B = 4
_CW = 2048
_E = 1024
_ROWS = _B * _CW
_TPW = _CW // _NW
_C = 32
_NP = _TPW // _C
_LANE = 16
_COLS = _E // _LANE


def _emb_body(x_hbm, wte_hbm, wpe_hbm, out_hbm,
              pos_v, tok0_v, tok1_v, idx0_v, idx1_v,
              sg0, sg1, ss0, ss1):
    wid = lax.axis_index("s") * _NC + lax.axis_index("c")
    t0 = wid * _TPW
    toks = (tok0_v, tok1_v)
    idxs = (idx0_v, idx1_v)
    gsems = (sg0, sg1)
    ssems = (ss0, ss1)
    chunks = [(p, b) for p in range(_NP) for b in range(_B)]
    n = len(chunks)

    def start_gather(i):
        p, b = chunks[i]
        s = i % 2
        pltpu.sync_copy(x_hbm.at[pl.ds(b * _CW + t0 + p * _C, _C)], idxs[s])
        return pltpu.async_copy(wte_hbm.at[idxs[s]], toks[s], gsems[s])

    pltpu.sync_copy(wpe_hbm.at[pl.ds(t0, _C)], pos_v)
    gather = start_gather(0)
    scatters = [None, None]
    for i in range(n):
        p, b = chunks[i]
        s = i % 2
        nxt = (i + 1) % 2
        if i + 1 < n:
            if scatters[nxt] is not None:
                scatters[nxt].wait()
                scatters[nxt] = None
            next_gather = start_gather(i + 1)
        gather.wait()
        if i + 1 < n:
            gather = next_gather
        if p > 0 and b == 0:
            pltpu.sync_copy(wpe_hbm.at[pl.ds(t0 + p * _C, _C)], pos_v)
        tok = toks[s]

        @plsc.parallel_loop(0, _C, unroll=2)
        def add_row(r, tok=tok):
            for j in range(_COLS):
                plsc.addupdate(tok.at[r, pl.ds(j * _LANE, _LANE)],
                               pos_v[r, pl.ds(j * _LANE, _LANE)])
        row0 = b * _CW + t0 + p * _C
        scatters[s] = pltpu.async_copy(tok, out_hbm.at[pl.ds(row0, _C)],
                                       ssems[s])
    for d in scatters:
        if d is not None:
            d.wait()


@jax.jit
def _emb(x_flat, wte, wpe):
    mesh = plsc.VectorSubcoreMesh(core_axis_name="c", subcore_axis_name="s")
    return pl.kernel(
        _emb_body,
        out_type=jax.ShapeDtypeStruct((_ROWS, _E), jnp.float32),
        mesh=mesh,
        scratch_types=[
            pltpu.VMEM((_C, _E), jnp.float32),
            pltpu.VMEM((_C, _E), jnp.float32),
            pltpu.VMEM((_C, _E), jnp.float32),
            pltpu.VMEM((_C,), jnp.int32),
            pltpu.VMEM((_C,), jnp.int32),
            pltpu.SemaphoreType.DMA,
            pltpu.SemaphoreType.DMA,
            pltpu.SemaphoreType.DMA,
            pltpu.SemaphoreType.DMA,
        ],
    )(x_flat, wte, wpe)


def kernel(x, wte, wpe, pos):
    del pos
    x_flat = x.reshape(_ROWS).astype(jnp.int32)
    out = _emb(x_flat, wte, wpe)
    return out.reshape(_B, _CW, _E)


# speedup vs baseline: 1.4112x; 1.0337x over previous
import jax
import jax.numpy as jnp
from jax import lax
from jax.experimental import pallas as pl
from jax.experimental.pallas import tpu as pltpu
from jax.experimental.pallas import tpu_sc as plsc

_INFO = plsc.get_sparse_core_info()
_NC, _NS = _INFO.num_cores, _INFO.num_subcores
_NW = _NC * _NS

_B = 4
_CW = 2048
_E = 1024
_ROWS = _B * _CW
_TPW = _CW // _NW
_C = 32
_NCHUNK = _TPW // _C * _B
_LANE = 16
_COLS = _E // _LANE


def _emb_body(x_hbm, wte_hbm, wpe_hbm, out_hbm,
              pos_v, tok0_v, tok1_v, idx0_v, idx1_v,
              sg0, sg1, ss0, ss1):
    wid = lax.axis_index("s") * _NC + lax.axis_index("c")
    t0 = wid * _TPW
    toks = (tok0_v, tok1_v)
    idxs = (idx0_v, idx1_v)
    gsems = (sg0, sg1)
    ssems = (ss0, ss1)

    def idx_off(i):
        return (i & 3) * _CW + t0 + (i >> 2) * _C

    def start_gather(i, s):
        pltpu.sync_copy(x_hbm.at[pl.ds(idx_off(i), _C)], idxs[s])
        pltpu.async_copy(wte_hbm.at[idxs[s]], toks[s], gsems[s])

    pltpu.sync_copy(wpe_hbm.at[pl.ds(t0, _C)], pos_v)
    start_gather(0, 0)

    def group(g, carry):
        for s in range(2):
            i = 2 * g + s
            o = s ^ 1
            @pl.when(i >= 1)
            def _():
                pltpu.make_async_copy(
                    toks[o], out_hbm.at[pl.ds(idx_off(i - 1), _C)], ssems[o]
                ).wait()

            @pl.when(i <= _NCHUNK - 2)
            def _():
                start_gather(i + 1, o)

            pltpu.make_async_copy(
                wte_hbm.at[idxs[s]], toks[s], gsems[s]).wait()

            @pl.when(i == _B)
            def _():
                pltpu.sync_copy(wpe_hbm.at[pl.ds(t0 + _C, _C)], pos_v)

            tok = toks[s]

            @plsc.parallel_loop(0, _C, unroll=8)
            def add_row(r, tok=tok):
                for j in range(_COLS):
                    plsc.addupdate(tok.at[r, pl.ds(j * _LANE, _LANE)],
                                   pos_v[r, pl.ds(j * _LANE, _LANE)])

            pltpu.async_copy(tok, out_hbm.at[pl.ds(idx_off(i), _C)], ssems[s])
        return carry

    lax.fori_loop(0, _NCHUNK // 2, group, 0)
    pltpu.make_async_copy(
        toks[1], out_hbm.at[pl.ds(idx_off(_NCHUNK - 1), _C)], ssems[1]
    ).wait()


@jax.jit
def _emb(x_flat, wte, wpe):
    mesh = plsc.VectorSubcoreMesh(core_axis_name="c", subcore_axis_name="s")
    return pl.kernel(
        _emb_body,
        out_type=jax.ShapeDtypeStruct((_ROWS, _E), jnp.float32),
        mesh=mesh,
        scratch_types=[
            pltpu.VMEM((_C, _E), jnp.float32),
            pltpu.VMEM((_C, _E), jnp.float32),
            pltpu.VMEM((_C, _E), jnp.float32),
            pltpu.VMEM((_C,), jnp.int32),
            pltpu.VMEM((_C,), jnp.int32),
            pltpu.SemaphoreType.DMA,
            pltpu.SemaphoreType.DMA,
            pltpu.SemaphoreType.DMA,
            pltpu.SemaphoreType.DMA,
        ],
    )(x_flat, wte, wpe)


def kernel(x, wte, wpe, pos):
    del pos
    x_flat = x.reshape(_ROWS).astype(jnp.int32)
    out = _emb(x_flat, wte, wpe)
    return out.reshape(_B, _CW, _E)


# speedup vs baseline: 1.5298x; 1.0840x over previous
import jax
import jax.numpy as jnp
from jax import lax
from jax.experimental import pallas as pl
from jax.experimental.pallas import tpu as pltpu
from jax.experimental.pallas import tpu_sc as plsc

_INFO = plsc.get_sparse_core_info()
_NC, _NS = _INFO.num_cores, _INFO.num_subcores
_NW = _NC * _NS

_B = 4
_CW = 2048
_E = 1024
_ROWS = _B * _CW
_TPW = _CW // _NW
_C = 32
_NP = _TPW // _C
_NCHUNK = _NP * _B
_LANE = 16
_COLS = _E // _LANE


def _emb_body(x_hbm, wte_hbm, wpe_hbm, out_hbm,
              pos_v, tok0_v, tok1_v, idx_v,
              sg0, sg1, ss0, ss1, sidx):
    wid = lax.axis_index("s") * _NC + lax.axis_index("c")
    t0 = wid * _TPW
    toks = (tok0_v, tok1_v)
    gsems = (sg0, sg1)
    ssems = (ss0, ss1)

    def out_off(i):
        return (i & 3) * _CW + t0 + (i >> 2) * _C

    descs = [
        pltpu.async_copy(
            x_hbm.at[pl.ds(out_off(i), _C)],
            idx_v.at[pl.ds(i * _C, _C)],
            sidx,
        )
        for i in range(_NCHUNK)
    ]
    for d in descs:
        d.wait()

    def start_gather(i, s):
        pltpu.async_copy(
            wte_hbm.at[idx_v.at[pl.ds(i * _C, _C)]], toks[s], gsems[s])

    start_gather(0, 0)
    pltpu.sync_copy(wpe_hbm.at[pl.ds(t0, _C)], pos_v)

    def group(g, carry):
        for s in range(2):
            i = 2 * g + s
            o = s ^ 1
            @pl.when(i >= 1)
            def _():
                pltpu.make_async_copy(
                    toks[o], out_hbm.at[pl.ds(out_off(i - 1), _C)], ssems[o]
                ).wait()

            @pl.when(i <= _NCHUNK - 2)
            def _():
                start_gather(i + 1, o)

            pltpu.make_async_copy(
                wte_hbm.at[idx_v.at[pl.ds(i * _C, _C)]], toks[s],
                gsems[s]).wait()

            @pl.when(i == _B)
            def _():
                pltpu.sync_copy(wpe_hbm.at[pl.ds(t0 + _C, _C)], pos_v)

            tok = toks[s]

            @plsc.parallel_loop(0, _C, unroll=8)
            def add_row(r, tok=tok):
                for j in range(_COLS):
                    plsc.addupdate(tok.at[r, pl.ds(j * _LANE, _LANE)],
                                   pos_v[r, pl.ds(j * _LANE, _LANE)])

            pltpu.async_copy(tok, out_hbm.at[pl.ds(out_off(i), _C)], ssems[s])
        return carry

    lax.fori_loop(0, _NCHUNK // 2, group, 0)
    pltpu.make_async_copy(
        toks[1], out_hbm.at[pl.ds(out_off(_NCHUNK - 1), _C)], ssems[1]
    ).wait()


@jax.jit
def _emb(x_flat, wte, wpe):
    mesh = plsc.VectorSubcoreMesh(core_axis_name="c", subcore_axis_name="s")
    return pl.kernel(
        _emb_body,
        out_type=jax.ShapeDtypeStruct((_ROWS, _E), jnp.float32),
        mesh=mesh,
        scratch_types=[
            pltpu.VMEM((_C, _E), jnp.float32),
            pltpu.VMEM((_C, _E), jnp.float32),
            pltpu.VMEM((_C, _E), jnp.float32),
            pltpu.VMEM((_NCHUNK * _C,), jnp.int32),
            pltpu.SemaphoreType.DMA,
            pltpu.SemaphoreType.DMA,
            pltpu.SemaphoreType.DMA,
            pltpu.SemaphoreType.DMA,
            pltpu.SemaphoreType.DMA,
        ],
    )(x_flat, wte, wpe)


def kernel(x, wte, wpe, pos):
    del pos
    out = _emb(x.reshape(_ROWS).astype(jnp.int32), wte, wpe)
    return out.reshape(_B, _CW, _E)


# speedup vs baseline: 1.6836x; 1.1005x over previous
import jax
import jax.numpy as jnp
from jax import lax
from jax.experimental import pallas as pl
from jax.experimental.pallas import tpu as pltpu
from jax.experimental.pallas import tpu_sc as plsc

_INFO = plsc.get_sparse_core_info()
_NC, _NS = _INFO.num_cores, _INFO.num_subcores
_NW = _NC * _NS

_B = 4
_CW = 2048
_E = 1024
_ROWS = _B * _CW
_TPW = _CW // _NW
_C = 16
_NB = 4
_NCHUNK = (_TPW // _C) * _B
_PC = 32
_LANE = 16
_COLS = _E // _LANE


def _emb_body(x_hbm, wte_hbm, wpe_hbm, out_hbm,
              pos_v, tok0_v, tok1_v, tok2_v, tok3_v, idx_v,
              sg0, sg1, sg2, sg3, ss0, ss1, ss2, ss3, sidx):
    wid = lax.axis_index("s") * _NC + lax.axis_index("c")
    t0 = wid * _TPW
    toks = (tok0_v, tok1_v, tok2_v, tok3_v)
    gsems = (sg0, sg1, sg2, sg3)
    ssems = (ss0, ss1, ss2, ss3)

    def out_off(i):
        return (i & 3) * _CW + t0 + (i >> 2) * _C

    descs = [
        pltpu.async_copy(
            x_hbm.at[pl.ds(out_off(i), _C)],
            idx_v.at[pl.ds(i * _C, _C)],
            sidx,
        )
        for i in range(_NCHUNK)
    ]
    for d in descs:
        d.wait()

    def start_gather(i, s):
        pltpu.async_copy(
            wte_hbm.at[idx_v.at[pl.ds(i * _C, _C)]], toks[s], gsems[s])

    start_gather(0, 0)
    pltpu.sync_copy(wpe_hbm.at[pl.ds(t0, _PC)], pos_v)

    def group(g, carry):
        for s in range(_NB):
            i = _NB * g + s
            nxt = (s + 1) % _NB
            @pl.when(i >= _NB - 1)
            def _():
                pltpu.make_async_copy(
                    toks[nxt],
                    out_hbm.at[pl.ds(out_off(i - (_NB - 1)), _C)],
                    ssems[nxt],
                ).wait()

            @pl.when(i <= _NCHUNK - 2)
            def _():
                start_gather(i + 1, nxt)

            pltpu.make_async_copy(
                wte_hbm.at[idx_v.at[pl.ds(i * _C, _C)]], toks[s],
                gsems[s]).wait()

            @pl.when(i == _NCHUNK // 2)
            def _():
                pltpu.sync_copy(wpe_hbm.at[pl.ds(t0 + _PC, _PC)], pos_v)

            tok = toks[s]
            pbase = ((i >> 2) & 1) * _C

            @plsc.parallel_loop(0, _C, unroll=4)
            def add_row(r, tok=tok, pbase=pbase):
                for j in range(_COLS):
                    plsc.addupdate(tok.at[r, pl.ds(j * _LANE, _LANE)],
                                   pos_v[pbase + r, pl.ds(j * _LANE, _LANE)])

            pltpu.async_copy(tok, out_hbm.at[pl.ds(out_off(i), _C)], ssems[s])
        return carry

    lax.fori_loop(0, _NCHUNK // _NB, group, 0)
    for i in range(_NCHUNK - (_NB - 1), _NCHUNK):
        pltpu.make_async_copy(
            toks[i % _NB], out_hbm.at[pl.ds(out_off(i), _C)], ssems[i % _NB]
        ).wait()


@jax.jit
def _emb(x_flat, wte, wpe):
    mesh = plsc.VectorSubcoreMesh(core_axis_name="c", subcore_axis_name="s")
    return pl.kernel(
        _emb_body,
        out_type=jax.ShapeDtypeStruct((_ROWS, _E), jnp.float32),
        mesh=mesh,
        scratch_types=[
            pltpu.VMEM((_PC, _E), jnp.float32),
            pltpu.VMEM((_C, _E), jnp.float32),
            pltpu.VMEM((_C, _E), jnp.float32),
            pltpu.VMEM((_C, _E), jnp.float32),
            pltpu.VMEM((_C, _E), jnp.float32),
            pltpu.VMEM((_NCHUNK * _C,), jnp.int32),
            pltpu.SemaphoreType.DMA,
            pltpu.SemaphoreType.DMA,
            pltpu.SemaphoreType.DMA,
            pltpu.SemaphoreType.DMA,
            pltpu.SemaphoreType.DMA,
            pltpu.SemaphoreType.DMA,
            pltpu.SemaphoreType.DMA,
            pltpu.SemaphoreType.DMA,
            pltpu.SemaphoreType.DMA,
        ],
    )(x_flat, wte, wpe)


def kernel(x, wte, wpe, pos):
    del pos
    out = _emb(x.reshape(_ROWS).astype(jnp.int32), wte, wpe)
    return out.reshape(_B, _CW, _E)
